# baseline (speedup 1.0000x reference)
import functools

import jax
import jax.numpy as jnp
from jax import lax
from jax.experimental import pallas as pl
from jax.experimental.pallas import tpu as pltpu
from jax.experimental.pallas import tpu_sc as plsc

NEG_INF = float("-inf")


def _tc_linear(features, node, w1t, w2t):
    n, d_in = features.shape
    d_pos = node.shape[1]
    d_out = w1t.shape[1]
    b = 2000
    assert n % b == 0

    def body(f_ref, n_ref, w1_ref, w2_ref, z_ref, q_ref):
        q = n_ref[:, 0:1] * w2_ref[0:1, :]
        for k in range(1, d_pos):
            q += n_ref[:, k:k + 1] * w2_ref[k:k + 1, :]
        z_ref[...] = lax.dot_general(
            f_ref[...], w1_ref[...], (((1,), (0,)), ((), ())),
            precision=lax.Precision.HIGHEST) + q
        q_ref[...] = q

    return pl.pallas_call(
        body,
        grid=(n // b,),
        in_specs=[
            pl.BlockSpec((b, d_in), lambda i: (i, 0)),
            pl.BlockSpec((b, d_pos), lambda i: (i, 0)),
            pl.BlockSpec((d_in, d_out), lambda i: (0, 0)),
            pl.BlockSpec((d_pos, d_out), lambda i: (0, 0)),
        ],
        out_specs=[
            pl.BlockSpec((b, d_out), lambda i: (i, 0)),
            pl.BlockSpec((b, d_out), lambda i: (i, 0)),
        ],
        out_shape=[
            jax.ShapeDtypeStruct((n, d_out), jnp.float32),
            jax.ShapeDtypeStruct((n, d_out), jnp.float32),
        ],
    )(features, node, w1t, w2t)


def _tc_sub(m, q):
    n, d = m.shape
    b = 2000
    assert n % b == 0

    def body(m_ref, q_ref, o_ref):
        o_ref[...] = m_ref[...] - q_ref[...]

    return pl.pallas_call(
        body,
        grid=(n // b,),
        in_specs=[pl.BlockSpec((b, d), lambda i: (i, 0))] * 2,
        out_specs=pl.BlockSpec((b, d), lambda i: (i, 0)),
        out_shape=jax.ShapeDtypeStruct((n, d), jnp.float32),
    )(m, q)


def _sc_segmax(z, dst, src):
    n, d = z.shape
    e = dst.shape[0]
    try:
        info = plsc.get_sparse_core_info()
        nc, ns, l = info.num_cores, info.num_subcores, info.num_lanes
    except Exception:
        nc, ns, l = 2, 16, 16
    nw = nc * ns
    assert n % 8 == 0
    r8 = (n // nw) // 8 * 8
    a = (n - nw * r8) // 8
    assert nw * r8 + 8 * a == n and a <= nw
    rmax = r8 + (8 if a > 0 else 0)
    c = 2560
    g = 128
    nch = e // c
    assert e % c == 0 and c % l == 0 and g % l == 0 and d % l == 0

    mesh = plsc.VectorSubcoreMesh(core_axis_name="c", subcore_axis_name="s")

    @functools.partial(
        pl.kernel,
        mesh=mesh,
        compiler_params=pltpu.CompilerParams(needs_layout_passes=False),
        out_type=jax.ShapeDtypeStruct((n, d), jnp.float32),
        scratch_types=[
            pltpu.VMEM((rmax + 1, d), jnp.float32),
            pltpu.VMEM((rmax + 1, d), jnp.float32),
            pltpu.VMEM((2, c), jnp.int32),
            pltpu.VMEM((2, c), jnp.int32),
            pltpu.VMEM((2 * g + 16,), jnp.int32),
            pltpu.VMEM((2 * g + 16,), jnp.int32),
            pltpu.VMEM((2, g), jnp.int32),
            pltpu.VMEM((2, g), jnp.int32),
            pltpu.VMEM((2, g, d), jnp.float32),
            pltpu.SemaphoreType.DMA,
            pltpu.SemaphoreType.DMA,
            pltpu.SemaphoreType.DMA,
        ],
    )
    def segmax_kernel(z_hbm, dst_hbm, src_hbm, out_hbm,
                      acc, acc1, dstb, srcb, psrc, ploc, sidx, sloc, rows,
                      semd, sems, semg):
        wid = lax.axis_index("s") * nc + lax.axis_index("c")
        lo = wid * r8 + 8 * jnp.minimum(wid, a)
        hi = lo + jnp.where(wid < a, r8 + 8, r8)

        def init_row(i, _):
            for cb in range(d // l):
                acc[i, pl.ds(cb * l, l)] = jnp.full((l,), NEG_INF, jnp.float32)
                acc1[i, pl.ds(cb * l, l)] = jnp.full((l,), NEG_INF,
                                                     jnp.float32)
            return 0
        lax.fori_loop(0, rmax + 1, init_row, 0)

        def wait_rows(s):
            pltpu.make_async_copy(z_hbm.at[pl.ds(0, g)], rows.at[s],
                                  semg).wait()

        def accum_batch(s):
            def accum(jg, _):
                lv = sloc[s, pl.ds(jg * l, l)]
                for lane in range(l):
                    dl = lv[lane]
                    row = jg * l + lane
                    tgt = acc if lane % 2 == 0 else acc1
                    for cb in range(d // l):
                        sl = pl.ds(cb * l, l)
                        tgt[dl, sl] = jnp.maximum(tgt[dl, sl],
                                                  rows[s, row, sl])
                return 0
            lax.fori_loop(0, g // l, accum, 0)

        def stage_and_fire(s):
            for j in range(g // l):
                t = psrc[pl.ds(j * l, l)]
                sidx[s, pl.ds(j * l, l)] = t
                t = ploc[pl.ds(j * l, l)]
                sloc[s, pl.ds(j * l, l)] = t
            pltpu.async_copy(z_hbm.at[sidx.at[s]], rows.at[s], semg)

        gpc = c // l
        gu = 8
        assert gpc % gu == 0 and gu * l <= g

        pltpu.async_copy(dst_hbm.at[pl.ds(0, c)], dstb.at[0], semd)
        pltpu.async_copy(src_hbm.at[pl.ds(0, c)], srcb.at[0], sems)

        def chunk_body(ch, state):
            cur, nd = state
            buf = lax.rem(ch, 2)
            pltpu.make_async_copy(dst_hbm.at[pl.ds(ch * c, c)],
                                  dstb.at[buf], semd).wait()
            pltpu.make_async_copy(src_hbm.at[pl.ds(ch * c, c)],
                                  srcb.at[buf], sems).wait()

            @pl.when(ch + 1 < nch)
            def _():
                nb = lax.rem(ch + 1, 2)
                pltpu.async_copy(dst_hbm.at[pl.ds((ch + 1) * c, c)],
                                 dstb.at[nb], semd)
                pltpu.async_copy(src_hbm.at[pl.ds((ch + 1) * c, c)],
                                 srcb.at[nb], sems)

            def outer(o, state):
                cur, nd = state
                dvs, svs, msks, prefs = [], [], [], []
                for u in range(gu):
                    jj = o * gu + u
                    dv = dstb[buf, pl.ds(jj * l, l)]
                    sv = srcb[buf, pl.ds(jj * l, l)]
                    msk = (dv >= lo) & (dv < hi)
                    pref = plsc.cumsum(
                        jnp.where(msk, jnp.ones((l,), jnp.int32),
                                  jnp.zeros((l,), jnp.int32)))
                    dvs.append(dv)
                    svs.append(sv)
                    msks.append(msk)
                    prefs.append(pref)
                curs = [cur]
                for u in range(gu):
                    curs.append(curs[-1] + prefs[u][l - 1])
                for u in range(gu):
                    pos = curs[u] + prefs[u] - 1
                    plsc.store_scatter(psrc, [pos], svs[u], mask=msks[u])
                    plsc.store_scatter(ploc, [pos], dvs[u] - lo,
                                       mask=msks[u])
                cur = curs[gu]

                @pl.when(cur >= g)
                def _():
                    s = lax.rem(nd, 2)
                    stage_and_fire(s)
                    @pl.when(nd >= 1)
                    def _():
                        sp = lax.rem(nd + 1, 2)
                        wait_rows(sp)
                        accum_batch(sp)
                    for j in range(g // l):
                        t = psrc[pl.ds(g + j * l, l)]
                        psrc[pl.ds(j * l, l)] = t
                        t = ploc[pl.ds(g + j * l, l)]
                        ploc[pl.ds(j * l, l)] = t
                nd = nd + jnp.where(cur >= g, 1, 0)
                return (lax.rem(cur, g), nd)
            return lax.fori_loop(0, gpc // gu, outer, (cur, nd))
        cur_end, nd_end = lax.fori_loop(0, nch, chunk_body, (0, 0))

        @pl.when(nd_end >= 1)
        def _():
            sp = lax.rem(nd_end + 1, 2)
            wait_rows(sp)
            accum_batch(sp)

        for j in range(g // l):
            psrc[pl.ds(cur_end + j * l, l)] = jnp.zeros((l,), jnp.int32)
            ploc[pl.ds(cur_end + j * l, l)] = jnp.full((l,), rmax, jnp.int32)
        sf = lax.rem(nd_end, 2)
        stage_and_fire(sf)
        wait_rows(sf)
        accum_batch(sf)

        def merge_row(i, _):
            for cb in range(d // l):
                sl = pl.ds(cb * l, l)
                acc[i, sl] = jnp.maximum(acc[i, sl], acc1[i, sl])
            return 0
        lax.fori_loop(0, rmax, merge_row, 0)

        @pl.when(wid < a)
        def _():
            pltpu.sync_copy(acc.at[pl.ds(0, r8 + 8)],
                            out_hbm.at[pl.ds(lo, r8 + 8)])

        @pl.when(wid >= a)
        def _():
            pltpu.sync_copy(acc.at[pl.ds(0, r8)], out_hbm.at[pl.ds(lo, r8)])

    return segmax_kernel(z, dst, src)


def kernel(node, features, edges, W):
    node = node.astype(jnp.float32)
    features = features.astype(jnp.float32)
    d_in = features.shape[1]
    w1t = W[:, :d_in].T
    w2t = W[:, d_in:].T
    dst = edges[:, 0].astype(jnp.int32)
    src = edges[:, 1].astype(jnp.int32)
    z, q = _tc_linear(features, node, w1t, w2t)
    m = _sc_segmax(z, dst, src)
    return _tc_sub(m, q)


# speedup vs baseline: 1.0056x; 1.0056x over previous
import functools

import jax
import jax.numpy as jnp
from jax import lax
from jax.experimental import pallas as pl
from jax.experimental.pallas import tpu as pltpu
from jax.experimental.pallas import tpu_sc as plsc

NEG_INF = float("-inf")


def _tc_linear(features, node, w1t, w2t):
    n, d_in = features.shape
    d_pos = node.shape[1]
    d_out = w1t.shape[1]
    b = 2000
    assert n % b == 0

    def body(f_ref, n_ref, w1_ref, w2_ref, z_ref, q_ref):
        q = n_ref[:, 0:1] * w2_ref[0:1, :]
        for k in range(1, d_pos):
            q += n_ref[:, k:k + 1] * w2_ref[k:k + 1, :]
        z_ref[...] = lax.dot_general(
            f_ref[...], w1_ref[...], (((1,), (0,)), ((), ())),
            precision=lax.Precision.HIGHEST) + q
        q_ref[...] = q

    return pl.pallas_call(
        body,
        grid=(n // b,),
        in_specs=[
            pl.BlockSpec((b, d_in), lambda i: (i, 0)),
            pl.BlockSpec((b, d_pos), lambda i: (i, 0)),
            pl.BlockSpec((d_in, d_out), lambda i: (0, 0)),
            pl.BlockSpec((d_pos, d_out), lambda i: (0, 0)),
        ],
        out_specs=[
            pl.BlockSpec((b, d_out), lambda i: (i, 0)),
            pl.BlockSpec((b, d_out), lambda i: (i, 0)),
        ],
        out_shape=[
            jax.ShapeDtypeStruct((n, d_out), jnp.float32),
            jax.ShapeDtypeStruct((n, d_out), jnp.float32),
        ],
    )(features, node, w1t, w2t)


def _tc_sub(m, q):
    n, d = m.shape
    b = 2000
    assert n % b == 0

    def body(m_ref, q_ref, o_ref):
        o_ref[...] = m_ref[...] - q_ref[...]

    return pl.pallas_call(
        body,
        grid=(n // b,),
        in_specs=[pl.BlockSpec((b, d), lambda i: (i, 0))] * 2,
        out_specs=pl.BlockSpec((b, d), lambda i: (i, 0)),
        out_shape=jax.ShapeDtypeStruct((n, d), jnp.float32),
    )(m, q)


def _sc_segmax(z, dst, src):
    n, d = z.shape
    e = dst.shape[0]
    try:
        info = plsc.get_sparse_core_info()
        nc, ns, l = info.num_cores, info.num_subcores, info.num_lanes
    except Exception:
        nc, ns, l = 2, 16, 16
    nw = nc * ns
    assert n % 8 == 0
    r8 = (n // nw) // 8 * 8
    a = (n - nw * r8) // 8
    assert nw * r8 + 8 * a == n and a <= nw
    rmax = r8 + (8 if a > 0 else 0)
    c = 2560
    g = 128
    nch = e // c
    assert e % c == 0 and c % l == 0 and g % l == 0 and d % l == 0

    mesh = plsc.VectorSubcoreMesh(core_axis_name="c", subcore_axis_name="s")

    @functools.partial(
        pl.kernel,
        mesh=mesh,
        compiler_params=pltpu.CompilerParams(needs_layout_passes=False),
        out_type=jax.ShapeDtypeStruct((n, d), jnp.float32),
        scratch_types=[
            pltpu.VMEM((rmax + 1, d), jnp.float32),
            pltpu.VMEM((2, c), jnp.int32),
            pltpu.VMEM((2, c), jnp.int32),
            pltpu.VMEM((2 * g + 16,), jnp.int32),
            pltpu.VMEM((2 * g + 16,), jnp.int32),
            pltpu.VMEM((4, g), jnp.int32),
            pltpu.VMEM((4, g), jnp.int32),
            pltpu.VMEM((4, g, d), jnp.float32),
            pltpu.SemaphoreType.DMA,
            pltpu.SemaphoreType.DMA,
            pltpu.SemaphoreType.DMA,
        ],
    )
    def segmax_kernel(z_hbm, dst_hbm, src_hbm, out_hbm,
                      acc, dstb, srcb, psrc, ploc, sidx, sloc, rows,
                      semd, sems, semg):
        wid = lax.axis_index("s") * nc + lax.axis_index("c")
        lo = wid * r8 + 8 * jnp.minimum(wid, a)
        hi = lo + jnp.where(wid < a, r8 + 8, r8)

        def init_row(i, _):
            for cb in range(d // l):
                acc[i, pl.ds(cb * l, l)] = jnp.full((l,), NEG_INF, jnp.float32)
            return 0
        lax.fori_loop(0, rmax + 1, init_row, 0)

        def wait_rows(s):
            pltpu.make_async_copy(z_hbm.at[pl.ds(0, g)], rows.at[s],
                                  semg).wait()

        def accum_batch(s):
            def accum(jg, _):
                lv = sloc[s, pl.ds(jg * l, l)]
                for lane in range(l):
                    dl = lv[lane]
                    row = jg * l + lane
                    for cb in range(d // l):
                        sl = pl.ds(cb * l, l)
                        acc[dl, sl] = jnp.maximum(acc[dl, sl],
                                                  rows[s, row, sl])
                return 0
            lax.fori_loop(0, g // l, accum, 0)

        def stage_and_fire(s):
            for j in range(g // l):
                t = psrc[pl.ds(j * l, l)]
                sidx[s, pl.ds(j * l, l)] = t
                t = ploc[pl.ds(j * l, l)]
                sloc[s, pl.ds(j * l, l)] = t
            pltpu.async_copy(z_hbm.at[sidx.at[s]], rows.at[s], semg)

        gpc = c // l
        gu = 8
        assert gpc % gu == 0 and gu * l <= g

        pltpu.async_copy(dst_hbm.at[pl.ds(0, c)], dstb.at[0], semd)
        pltpu.async_copy(src_hbm.at[pl.ds(0, c)], srcb.at[0], sems)

        def chunk_body(ch, state):
            cur, nd = state
            buf = lax.rem(ch, 2)
            pltpu.make_async_copy(dst_hbm.at[pl.ds(ch * c, c)],
                                  dstb.at[buf], semd).wait()
            pltpu.make_async_copy(src_hbm.at[pl.ds(ch * c, c)],
                                  srcb.at[buf], sems).wait()

            @pl.when(ch + 1 < nch)
            def _():
                nb = lax.rem(ch + 1, 2)
                pltpu.async_copy(dst_hbm.at[pl.ds((ch + 1) * c, c)],
                                 dstb.at[nb], semd)
                pltpu.async_copy(src_hbm.at[pl.ds((ch + 1) * c, c)],
                                 srcb.at[nb], sems)

            def outer(o, state):
                cur, nd = state
                dvs, svs, msks, prefs = [], [], [], []
                for u in range(gu):
                    jj = o * gu + u
                    dv = dstb[buf, pl.ds(jj * l, l)]
                    sv = srcb[buf, pl.ds(jj * l, l)]
                    msk = (dv >= lo) & (dv < hi)
                    pref = plsc.cumsum(
                        jnp.where(msk, jnp.ones((l,), jnp.int32),
                                  jnp.zeros((l,), jnp.int32)))
                    dvs.append(dv)
                    svs.append(sv)
                    msks.append(msk)
                    prefs.append(pref)
                curs = [cur]
                for u in range(gu):
                    curs.append(curs[-1] + prefs[u][l - 1])
                for u in range(gu):
                    pos = curs[u] + prefs[u] - 1
                    plsc.store_scatter(psrc, [pos], svs[u], mask=msks[u])
                    plsc.store_scatter(ploc, [pos], dvs[u] - lo,
                                       mask=msks[u])
                cur = curs[gu]

                @pl.when(cur >= g)
                def _():
                    s = lax.rem(nd, 4)
                    stage_and_fire(s)
                    @pl.when(nd >= 3)
                    def _():
                        sp = lax.rem(nd + 1, 4)
                        wait_rows(sp)
                        accum_batch(sp)
                    for j in range(g // l):
                        t = psrc[pl.ds(g + j * l, l)]
                        psrc[pl.ds(j * l, l)] = t
                        t = ploc[pl.ds(g + j * l, l)]
                        ploc[pl.ds(j * l, l)] = t
                nd = nd + jnp.where(cur >= g, 1, 0)
                return (lax.rem(cur, g), nd)
            return lax.fori_loop(0, gpc // gu, outer, (cur, nd))
        cur_end, nd_end = lax.fori_loop(0, nch, chunk_body, (0, 0))

        for j in (3, 2, 1):
            @pl.when(nd_end >= j)
            def _(j=j):
                sp = lax.rem(nd_end + (4 - j), 4)
                wait_rows(sp)
                accum_batch(sp)

        for j in range(g // l):
            psrc[pl.ds(cur_end + j * l, l)] = jnp.zeros((l,), jnp.int32)
            ploc[pl.ds(cur_end + j * l, l)] = jnp.full((l,), rmax, jnp.int32)
        sf = lax.rem(nd_end, 4)
        stage_and_fire(sf)
        wait_rows(sf)
        accum_batch(sf)

        @pl.when(wid < a)
        def _():
            pltpu.sync_copy(acc.at[pl.ds(0, r8 + 8)],
                            out_hbm.at[pl.ds(lo, r8 + 8)])

        @pl.when(wid >= a)
        def _():
            pltpu.sync_copy(acc.at[pl.ds(0, r8)], out_hbm.at[pl.ds(lo, r8)])

    return segmax_kernel(z, dst, src)


def kernel(node, features, edges, W):
    node = node.astype(jnp.float32)
    features = features.astype(jnp.float32)
    d_in = features.shape[1]
    w1t = W[:, :d_in].T
    w2t = W[:, d_in:].T
    dst = edges[:, 0].astype(jnp.int32)
    src = edges[:, 1].astype(jnp.int32)
    z, q = _tc_linear(features, node, w1t, w2t)
    m = _sc_segmax(z, dst, src)
    return _tc_sub(m, q)


# speedup vs baseline: 1.0697x; 1.0638x over previous
import functools

import jax
import jax.numpy as jnp
from jax import lax
from jax.experimental import pallas as pl
from jax.experimental.pallas import tpu as pltpu
from jax.experimental.pallas import tpu_sc as plsc

NEG_INF = float("-inf")


def _tc_linear(features, node, w1t, w2t):
    n, d_in = features.shape
    d_pos = node.shape[1]
    d_out = w1t.shape[1]
    b = 2000
    assert n % b == 0

    def body(f_ref, n_ref, w1_ref, w2_ref, z_ref, q_ref):
        q = n_ref[:, 0:1] * w2_ref[0:1, :]
        for k in range(1, d_pos):
            q += n_ref[:, k:k + 1] * w2_ref[k:k + 1, :]
        z_ref[...] = lax.dot_general(
            f_ref[...], w1_ref[...], (((1,), (0,)), ((), ())),
            precision=lax.Precision.HIGHEST) + q
        q_ref[...] = q

    return pl.pallas_call(
        body,
        grid=(n // b,),
        in_specs=[
            pl.BlockSpec((b, d_in), lambda i: (i, 0)),
            pl.BlockSpec((b, d_pos), lambda i: (i, 0)),
            pl.BlockSpec((d_in, d_out), lambda i: (0, 0)),
            pl.BlockSpec((d_pos, d_out), lambda i: (0, 0)),
        ],
        out_specs=[
            pl.BlockSpec((b, d_out), lambda i: (i, 0)),
            pl.BlockSpec((b, d_out), lambda i: (i, 0)),
        ],
        out_shape=[
            jax.ShapeDtypeStruct((n, d_out), jnp.float32),
            jax.ShapeDtypeStruct((n, d_out), jnp.float32),
        ],
    )(features, node, w1t, w2t)


def _tc_sub(m0, m1, q):
    n, d = m0.shape
    b = 2000
    assert n % b == 0

    def body(m0_ref, m1_ref, q_ref, o_ref):
        o_ref[...] = jnp.maximum(m0_ref[...], m1_ref[...]) - q_ref[...]

    return pl.pallas_call(
        body,
        grid=(n // b,),
        in_specs=[pl.BlockSpec((b, d), lambda i: (i, 0))] * 3,
        out_specs=pl.BlockSpec((b, d), lambda i: (i, 0)),
        out_shape=jax.ShapeDtypeStruct((n, d), jnp.float32),
    )(m0, m1, q)


def _sc_segmax(z, dst, src):
    n, d = z.shape
    e = dst.shape[0]
    try:
        info = plsc.get_sparse_core_info()
        nc, ns, l = info.num_cores, info.num_subcores, info.num_lanes
    except Exception:
        nc, ns, l = 2, 16, 16
    nw = nc * ns
    nr = nw // 2
    eh = e // 2
    assert n % 8 == 0 and e % 2 == 0
    r8 = (n // nr) // 8 * 8
    a = (n - nr * r8) // 8
    assert nr * r8 + 8 * a == n and a <= nr
    rmax = r8 + (8 if a > 0 else 0)
    c = 1280
    g = 128
    nch = eh // c
    assert eh % c == 0 and c % l == 0 and g % l == 0 and d % l == 0

    mesh = plsc.VectorSubcoreMesh(core_axis_name="c", subcore_axis_name="s")

    @functools.partial(
        pl.kernel,
        mesh=mesh,
        compiler_params=pltpu.CompilerParams(needs_layout_passes=False),
        out_type=jax.ShapeDtypeStruct((2, n, d), jnp.float32),
        scratch_types=[
            pltpu.VMEM((rmax + 1, d), jnp.float32),
            pltpu.VMEM((2, c), jnp.int32),
            pltpu.VMEM((2, c), jnp.int32),
            pltpu.VMEM((2 * g + 16,), jnp.int32),
            pltpu.VMEM((2 * g + 16,), jnp.int32),
            pltpu.VMEM((2, g), jnp.int32),
            pltpu.VMEM((2, g), jnp.int32),
            pltpu.VMEM((2, g, d), jnp.float32),
            pltpu.SemaphoreType.DMA,
            pltpu.SemaphoreType.DMA,
            pltpu.SemaphoreType.DMA,
        ],
    )
    def segmax_kernel(z_hbm, dst_hbm, src_hbm, out_hbm,
                      acc, dstb, srcb, psrc, ploc, sidx, sloc, rows,
                      semd, sems, semg):
        wid = lax.axis_index("s") * nc + lax.axis_index("c")
        rid = lax.rem(wid, nr)
        h = wid // nr
        ebase = h * eh
        lo = rid * r8 + 8 * jnp.minimum(rid, a)
        hi = lo + jnp.where(rid < a, r8 + 8, r8)

        def init_row(i, _):
            for cb in range(d // l):
                acc[i, pl.ds(cb * l, l)] = jnp.full((l,), NEG_INF, jnp.float32)
            return 0
        lax.fori_loop(0, rmax + 1, init_row, 0)

        def wait_rows(s):
            pltpu.make_async_copy(z_hbm.at[pl.ds(0, g)], rows.at[s],
                                  semg).wait()

        def accum_batch(s):
            def accum(jg, _):
                lv = sloc[s, pl.ds(jg * l, l)]
                for lane in range(l):
                    dl = lv[lane]
                    row = jg * l + lane
                    for cb in range(d // l):
                        sl = pl.ds(cb * l, l)
                        acc[dl, sl] = jnp.maximum(acc[dl, sl],
                                                  rows[s, row, sl])
                return 0
            lax.fori_loop(0, g // l, accum, 0)

        def stage_and_fire(s):
            for j in range(g // l):
                t = psrc[pl.ds(j * l, l)]
                sidx[s, pl.ds(j * l, l)] = t
                t = ploc[pl.ds(j * l, l)]
                sloc[s, pl.ds(j * l, l)] = t
            pltpu.async_copy(z_hbm.at[sidx.at[s]], rows.at[s], semg)

        gpc = c // l
        gu = 8
        assert gpc % gu == 0 and gu * l <= g

        pltpu.async_copy(dst_hbm.at[pl.ds(ebase, c)], dstb.at[0], semd)
        pltpu.async_copy(src_hbm.at[pl.ds(ebase, c)], srcb.at[0], sems)

        def chunk_body(ch, state):
            cur, nd = state
            buf = lax.rem(ch, 2)
            pltpu.make_async_copy(dst_hbm.at[pl.ds(ebase + ch * c, c)],
                                  dstb.at[buf], semd).wait()
            pltpu.make_async_copy(src_hbm.at[pl.ds(ebase + ch * c, c)],
                                  srcb.at[buf], sems).wait()

            @pl.when(ch + 1 < nch)
            def _():
                nb = lax.rem(ch + 1, 2)
                pltpu.async_copy(dst_hbm.at[pl.ds(ebase + (ch + 1) * c, c)],
                                 dstb.at[nb], semd)
                pltpu.async_copy(src_hbm.at[pl.ds(ebase + (ch + 1) * c, c)],
                                 srcb.at[nb], sems)

            def outer(o, state):
                cur, nd = state
                dvs, svs, msks, prefs = [], [], [], []
                for u in range(gu):
                    jj = o * gu + u
                    dv = dstb[buf, pl.ds(jj * l, l)]
                    sv = srcb[buf, pl.ds(jj * l, l)]
                    msk = (dv >= lo) & (dv < hi)
                    pref = plsc.cumsum(
                        jnp.where(msk, jnp.ones((l,), jnp.int32),
                                  jnp.zeros((l,), jnp.int32)))
                    dvs.append(dv)
                    svs.append(sv)
                    msks.append(msk)
                    prefs.append(pref)
                curs = [cur]
                for u in range(gu):
                    curs.append(curs[-1] + prefs[u][l - 1])
                for u in range(gu):
                    pos = curs[u] + prefs[u] - 1
                    plsc.store_scatter(psrc, [pos], svs[u], mask=msks[u])
                    plsc.store_scatter(ploc, [pos], dvs[u] - lo,
                                       mask=msks[u])
                cur = curs[gu]

                @pl.when(cur >= g)
                def _():
                    s = lax.rem(nd, 2)
                    stage_and_fire(s)
                    @pl.when(nd >= 1)
                    def _():
                        sp = lax.rem(nd + 1, 2)
                        wait_rows(sp)
                        accum_batch(sp)
                    for j in range(g // l):
                        t = psrc[pl.ds(g + j * l, l)]
                        psrc[pl.ds(j * l, l)] = t
                        t = ploc[pl.ds(g + j * l, l)]
                        ploc[pl.ds(j * l, l)] = t
                nd = nd + jnp.where(cur >= g, 1, 0)
                return (lax.rem(cur, g), nd)
            return lax.fori_loop(0, gpc // gu, outer, (cur, nd))
        cur_end, nd_end = lax.fori_loop(0, nch, chunk_body, (0, 0))

        @pl.when(nd_end >= 1)
        def _():
            sp = lax.rem(nd_end + 1, 2)
            wait_rows(sp)
            accum_batch(sp)

        for j in range(g // l):
            psrc[pl.ds(cur_end + j * l, l)] = jnp.zeros((l,), jnp.int32)
            ploc[pl.ds(cur_end + j * l, l)] = jnp.full((l,), rmax, jnp.int32)
        sf = lax.rem(nd_end, 2)
        stage_and_fire(sf)
        wait_rows(sf)
        accum_batch(sf)

        @pl.when(rid < a)
        def _():
            pltpu.sync_copy(acc.at[pl.ds(0, r8 + 8)],
                            out_hbm.at[h, pl.ds(lo, r8 + 8)])

        @pl.when(rid >= a)
        def _():
            pltpu.sync_copy(acc.at[pl.ds(0, r8)],
                            out_hbm.at[h, pl.ds(lo, r8)])

    return segmax_kernel(z, dst, src)


def kernel(node, features, edges, W):
    node = node.astype(jnp.float32)
    features = features.astype(jnp.float32)
    d_in = features.shape[1]
    w1t = W[:, :d_in].T
    w2t = W[:, d_in:].T
    dst = edges[:, 0].astype(jnp.int32)
    src = edges[:, 1].astype(jnp.int32)
    z, q = _tc_linear(features, node, w1t, w2t)
    m = _sc_segmax(z, dst, src)
    return _tc_sub(m[0], m[1], q)
